# Initial kernel scaffold; baseline (speedup 1.0000x reference)
import jax
import jax.numpy as jnp
from jax.experimental import pallas as pl


def kernel(edge_index, edge_weight, W, gamma1, beta1, gamma2, beta2):
    raise NotImplementedError("write your pallas kernel here")


# speedup vs baseline: 3.0417x; 3.0417x over previous
import functools

import jax
import jax.numpy as jnp
from jax import lax
from jax.experimental import pallas as pl
from jax.experimental.pallas import tpu as pltpu
from jax.experimental.pallas import tpu_sc as plsc

N = 10000
E = 320000
D = 128
EPS = 1e-5

NC = 2
NS = 16
NW = NC * NS

CHUNK = 256
EPW = 10240
E_PAD = EPW * NW
N_CHUNKS = EPW // CHUNK
N_SUB = CHUNK // 128
N_PAD = 10112
ROWS_PT = N_PAD // NS


def _spmm_kernel(src_hbm, dst_hbm, w_hbm, x_hbm, out_hbm,
                 accum, srcv, dstv, wv, rows, sem):
    cid = lax.axis_index("c")
    sid = lax.axis_index("s")
    wid = cid * NS + sid
    io16 = lax.iota(jnp.int32, 16)

    def _zero_row(i, c):
        for j in range(D // 16):
            rows[i, pl.ds(j * 16, 16)] = jnp.zeros((16,), jnp.float32)
        return c
    lax.fori_loop(0, CHUNK, _zero_row, 0)
    base = sid * ROWS_PT
    pltpu.sync_copy(rows.at[pl.ds(0, CHUNK)], accum.at[pl.ds(base, CHUNK)])
    pltpu.sync_copy(rows.at[pl.ds(0, CHUNK)],
                    accum.at[pl.ds(base + CHUNK, CHUNK)])
    pltpu.sync_copy(rows.at[pl.ds(0, ROWS_PT - 2 * CHUNK)],
                    accum.at[pl.ds(base + 2 * CHUNK, ROWS_PT - 2 * CHUNK)])
    plsc.subcore_barrier()

    idx_rows_per_chunk = CHUNK // 128
    idx_base = wid * (EPW // 128)
    w_rows_per_chunk = CHUNK // 16
    w_base = wid * (EPW // 16)

    def _chunk(k, c):
        rb = idx_base + k * idx_rows_per_chunk
        wb = w_base + k * w_rows_per_chunk
        pltpu.sync_copy(src_hbm.at[pl.ds(rb, idx_rows_per_chunk)], srcv)
        pltpu.sync_copy(dst_hbm.at[pl.ds(rb, idx_rows_per_chunk)], dstv)
        pltpu.sync_copy(w_hbm.at[pl.ds(wb, w_rows_per_chunk)], wv)
        cps = [pltpu.async_copy(x_hbm.at[srcv.at[s]],
                                rows.at[pl.ds(s * 128, 128)], sem)
               for s in range(N_SUB)]
        for cp in cps:
            cp.wait()

        def _grp(g, cc):
            w16 = wv[g]
            for r in range(16):
                ws = w16[r]
                row = g * 16 + r
                for j in range(D // 16):
                    rows[row, pl.ds(j * 16, 16)] = (
                        rows[row, pl.ds(j * 16, 16)] * ws)
            return cc
        lax.fori_loop(0, CHUNK // 16, _grp, 0)

        for s in range(N_SUB):
            pltpu.sync_copy(rows.at[pl.ds(s * 128, 128)],
                            accum.at[dstv.at[s]], add=True)
        return c
    lax.fori_loop(0, N_CHUNKS, _chunk, 0)

    plsc.subcore_barrier()
    pltpu.sync_copy(accum.at[pl.ds(base, ROWS_PT)],
                    out_hbm.at[cid, pl.ds(base, ROWS_PT)])


@jax.jit
def _sc_spmm_call(src2d, dst2d, w16d, x):
    mesh = plsc.VectorSubcoreMesh(core_axis_name="c", subcore_axis_name="s",
                                  num_cores=NC, num_subcores=NS)
    f = pl.kernel(
        _spmm_kernel,
        out_type=jax.ShapeDtypeStruct((NC, N_PAD, D), jnp.float32),
        mesh=mesh,
        scratch_types=[
            pltpu.VMEM_SHARED((N_PAD, D), jnp.float32),
            pltpu.VMEM((N_SUB, 128), jnp.int32),
            pltpu.VMEM((N_SUB, 128), jnp.int32),
            pltpu.VMEM((CHUNK // 16, 16), jnp.float32),
            pltpu.VMEM((CHUNK, D), jnp.float32),
            pltpu.SemaphoreType.DMA,
        ],
    )
    return f(src2d, dst2d, w16d, x)


def _bn1_body(p_ref, g_ref, b_ref, o_ref):
    x = p_ref[0, :N, :] + p_ref[1, :N, :]
    inv_n = jnp.float32(1.0 / N)
    mean = jnp.sum(x, axis=0, keepdims=True) * inv_n
    msq = jnp.sum(x * x, axis=0, keepdims=True) * inv_n
    var = msq - mean * mean
    inv = lax.rsqrt(var + EPS) * g_ref[...]
    o_ref[...] = (x - mean) * inv + b_ref[...]


def _bn2_body(p_ref, x1_ref, g_ref, b_ref, o_ref):
    x = p_ref[0, :N, :] + p_ref[1, :N, :]
    inv_n = jnp.float32(1.0 / N)
    mean = jnp.sum(x, axis=0, keepdims=True) * inv_n
    msq = jnp.sum(x * x, axis=0, keepdims=True) * inv_n
    var = msq - mean * mean
    inv = lax.rsqrt(var + EPS) * g_ref[...]
    y = (x - mean) * inv + b_ref[...]
    o_ref[...] = (x1_ref[...] + y) * jnp.float32(0.5)


@jax.jit
def _bn1(partials, gamma, beta):
    return pl.pallas_call(
        _bn1_body,
        out_shape=jax.ShapeDtypeStruct((N, D), jnp.float32),
    )(partials, gamma.reshape(1, D), beta.reshape(1, D))


@jax.jit
def _bn2(partials, x1, gamma, beta):
    return pl.pallas_call(
        _bn2_body,
        out_shape=jax.ShapeDtypeStruct((N, D), jnp.float32),
    )(partials, x1, gamma.reshape(1, D), beta.reshape(1, D))


def kernel(edge_index, edge_weight, W, gamma1, beta1, gamma2, beta2):
    src = edge_index[0].astype(jnp.int32)
    dst = edge_index[1].astype(jnp.int32)
    w = edge_weight.astype(jnp.float32)
    pad = E_PAD - E
    src = jnp.concatenate([src, jnp.zeros((pad,), jnp.int32)])
    dst = jnp.concatenate([dst, jnp.zeros((pad,), jnp.int32)])
    w = jnp.concatenate([w, jnp.zeros((pad,), jnp.float32)])
    src2d = src.reshape(E_PAD // 128, 128)
    dst2d = dst.reshape(E_PAD // 128, 128)
    w16d = w.reshape(E_PAD // 16, 16)

    p1 = _sc_spmm_call(src2d, dst2d, w16d, W)
    x1 = _bn1(p1, gamma1, beta1)
    p2 = _sc_spmm_call(src2d, dst2d, w16d, x1)
    return _bn2(p2, x1, gamma2, beta2)


# speedup vs baseline: 3.7071x; 1.2187x over previous
import jax
import jax.numpy as jnp
from jax import lax
from jax.experimental import pallas as pl
from jax.experimental.pallas import tpu as pltpu
from jax.experimental.pallas import tpu_sc as plsc

N = 10000
E = 320000
D = 128
EPS = 1e-5

NC = 2
NS = 16
NW = NC * NS

CHUNK = 128
EPW = 10240
E_PAD = EPW * NW
N_CHUNKS = EPW // CHUNK
WROWS = CHUNK // 16
N_PAD = 10112
ROWS_PT = N_PAD // NS
IDX_ROWS = E_PAD // 128
W_ROWS = E_PAD // 16


def _spmm_kernel(src_hbm, dst_hbm, w_hbm, x_hbm, out_hbm, accum,
                 srcv0, dstv0, wv0, rows0, semi0, semg0,
                 srcv1, dstv1, wv1, rows1, semi1, semg1):
    cid = lax.axis_index("c")
    sid = lax.axis_index("s")
    wid = cid * NS + sid
    bufs = ((srcv0, dstv0, wv0, rows0, semi0, semg0),
            (srcv1, dstv1, wv1, rows1, semi1, semg1))

    idx_base = wid * (EPW // 128)
    w_base = wid * (EPW // 16)

    def idx_issue(k, b):
        srcv, dstv, wv, _, semi, _ = bufs[b]
        rb = jnp.minimum(idx_base + k, IDX_ROWS - 1)
        wb = jnp.minimum(w_base + k * WROWS, W_ROWS - WROWS)
        pltpu.async_copy(src_hbm.at[pl.ds(rb, 1)], srcv, semi)
        pltpu.async_copy(dst_hbm.at[pl.ds(rb, 1)], dstv, semi)
        pltpu.async_copy(w_hbm.at[pl.ds(wb, WROWS)], wv, semi)

    def idx_wait(b):
        srcv, dstv, wv, _, semi, _ = bufs[b]
        pltpu.make_async_copy(src_hbm.at[pl.ds(0, 1)], srcv, semi).wait()
        pltpu.make_async_copy(dst_hbm.at[pl.ds(0, 1)], dstv, semi).wait()
        pltpu.make_async_copy(w_hbm.at[pl.ds(0, WROWS)], wv, semi).wait()

    def gather_issue(b):
        srcv, _, _, rows, _, semg = bufs[b]
        pltpu.async_copy(x_hbm.at[srcv.at[0]], rows, semg)

    def gather_wait(b):
        srcv, _, _, rows, _, semg = bufs[b]
        pltpu.make_async_copy(x_hbm.at[srcv.at[0]], rows, semg).wait()

    def _zero_row(i, c):
        for j in range(D // 16):
            rows0[i, pl.ds(j * 16, 16)] = jnp.zeros((16,), jnp.float32)
        return c
    lax.fori_loop(0, CHUNK, _zero_row, 0)
    base = sid * ROWS_PT
    for t in range(ROWS_PT // CHUNK):
        pltpu.sync_copy(rows0.at[pl.ds(0, CHUNK)],
                        accum.at[pl.ds(base + t * CHUNK, CHUNK)])
    rem = ROWS_PT % CHUNK
    pltpu.sync_copy(rows0.at[pl.ds(0, rem)],
                    accum.at[pl.ds(base + ROWS_PT - rem, rem)])
    plsc.subcore_barrier()

    idx_issue(0, 0)
    idx_wait(0)
    gather_issue(0)
    idx_issue(1, 1)

    def _super(k2, c):
        for j in range(2):
            k = k2 * 2 + j
            srcv, dstv, wv, rows, semi, semg = bufs[j]
            b1 = 1 - j
            gather_wait(j)
            idx_wait(b1)
            gather_issue(b1)

            def _grp(g, cc):
                w16 = wv[g]
                for r in range(16):
                    ws = w16[r]
                    row = g * 16 + r
                    for jj in range(D // 16):
                        rows[row, pl.ds(jj * 16, 16)] = (
                            rows[row, pl.ds(jj * 16, 16)] * ws)
                return cc
            lax.fori_loop(0, WROWS, _grp, 0)

            pltpu.sync_copy(rows, accum.at[dstv.at[0]], add=True)
            idx_issue(k + 2, j)
        return c
    lax.fori_loop(0, N_CHUNKS // 2, _super, 0)

    gather_wait(0)
    idx_wait(1)

    plsc.subcore_barrier()
    pltpu.sync_copy(accum.at[pl.ds(base, ROWS_PT)],
                    out_hbm.at[cid, pl.ds(base, ROWS_PT)])


@jax.jit
def _sc_spmm_call(src2d, dst2d, w16d, x):
    mesh = plsc.VectorSubcoreMesh(core_axis_name="c", subcore_axis_name="s",
                                  num_cores=NC, num_subcores=NS)
    buf_scratch = []
    for _ in range(2):
        buf_scratch += [
            pltpu.VMEM((1, 128), jnp.int32),
            pltpu.VMEM((1, 128), jnp.int32),
            pltpu.VMEM((WROWS, 16), jnp.float32),
            pltpu.VMEM((CHUNK, D), jnp.float32),
            pltpu.SemaphoreType.DMA,
            pltpu.SemaphoreType.DMA,
        ]
    f = pl.kernel(
        _spmm_kernel,
        out_type=jax.ShapeDtypeStruct((NC, N_PAD, D), jnp.float32),
        mesh=mesh,
        scratch_types=[pltpu.VMEM_SHARED((N_PAD, D), jnp.float32)]
        + buf_scratch,
    )
    return f(src2d, dst2d, w16d, x)


def _bn1_body(p_ref, g_ref, b_ref, o_ref):
    x = p_ref[0, :N, :] + p_ref[1, :N, :]
    inv_n = jnp.float32(1.0 / N)
    mean = jnp.sum(x, axis=0, keepdims=True) * inv_n
    msq = jnp.sum(x * x, axis=0, keepdims=True) * inv_n
    var = msq - mean * mean
    inv = lax.rsqrt(var + EPS) * g_ref[...]
    o_ref[...] = (x - mean) * inv + b_ref[...]


def _bn2_body(p_ref, x1_ref, g_ref, b_ref, o_ref):
    x = p_ref[0, :N, :] + p_ref[1, :N, :]
    inv_n = jnp.float32(1.0 / N)
    mean = jnp.sum(x, axis=0, keepdims=True) * inv_n
    msq = jnp.sum(x * x, axis=0, keepdims=True) * inv_n
    var = msq - mean * mean
    inv = lax.rsqrt(var + EPS) * g_ref[...]
    y = (x - mean) * inv + b_ref[...]
    o_ref[...] = (x1_ref[...] + y) * jnp.float32(0.5)


@jax.jit
def _bn1(partials, gamma, beta):
    return pl.pallas_call(
        _bn1_body,
        out_shape=jax.ShapeDtypeStruct((N, D), jnp.float32),
    )(partials, gamma.reshape(1, D), beta.reshape(1, D))


@jax.jit
def _bn2(partials, x1, gamma, beta):
    return pl.pallas_call(
        _bn2_body,
        out_shape=jax.ShapeDtypeStruct((N, D), jnp.float32),
    )(partials, x1, gamma.reshape(1, D), beta.reshape(1, D))


def kernel(edge_index, edge_weight, W, gamma1, beta1, gamma2, beta2):
    src = edge_index[0].astype(jnp.int32)
    dst = edge_index[1].astype(jnp.int32)
    w = edge_weight.astype(jnp.float32)
    pad = E_PAD - E
    src = jnp.concatenate([src, jnp.zeros((pad,), jnp.int32)])
    dst = jnp.concatenate([dst, jnp.zeros((pad,), jnp.int32)])
    w = jnp.concatenate([w, jnp.zeros((pad,), jnp.float32)])
    src2d = src.reshape(E_PAD // 128, 128)
    dst2d = dst.reshape(E_PAD // 128, 128)
    w16d = w.reshape(E_PAD // 16, 16)

    p1 = _sc_spmm_call(src2d, dst2d, w16d, W)
    x1 = _bn1(p1, gamma1, beta1)
    p2 = _sc_spmm_call(src2d, dst2d, w16d, x1)
    return _bn2(p2, x1, gamma2, beta2)


# speedup vs baseline: 3.7401x; 1.0089x over previous
import jax
import jax.numpy as jnp
from jax import lax
from jax.experimental import pallas as pl
from jax.experimental.pallas import tpu as pltpu
from jax.experimental.pallas import tpu_sc as plsc

N = 10000
E = 320000
D = 128
EPS = 1e-5

NC = 2
NS = 16
NW = NC * NS

CHUNK = 128
EPW = 10240
E_PAD = EPW * NW
N_CHUNKS = EPW // CHUNK
WROWS = CHUNK // 16
N_PAD = 10112
ROWS_PT = N_PAD // NS
IDX_ROWS = E_PAD // 128
W_ROWS = E_PAD // 16


def _spmm_kernel(src_hbm, dst_hbm, w_hbm, x_hbm, out_hbm, accum,
                 srcv0, dstv0, dstx0, wv0, rows0, semi0, semg0, sems0,
                 srcv1, dstv1, dstx1, wv1, rows1, semi1, semg1, sems1):
    cid = lax.axis_index("c")
    sid = lax.axis_index("s")
    wid = cid * NS + sid
    bufs = ((srcv0, dstv0, dstx0, wv0, rows0, semi0, semg0, sems0),
            (srcv1, dstv1, dstx1, wv1, rows1, semi1, semg1, sems1))

    idx_base = wid * (EPW // 128)
    w_base = wid * (EPW // 16)

    def idx_issue(k, b):
        srcv, dstv, _, wv, _, semi, _, _ = bufs[b]
        rb = jnp.minimum(idx_base + k, IDX_ROWS - 1)
        wb = jnp.minimum(w_base + k * WROWS, W_ROWS - WROWS)
        pltpu.async_copy(src_hbm.at[pl.ds(rb, 1)], srcv, semi)
        pltpu.async_copy(dst_hbm.at[pl.ds(rb, 1)], dstv, semi)
        pltpu.async_copy(w_hbm.at[pl.ds(wb, WROWS)], wv, semi)

    def idx_wait(b):
        srcv, dstv, _, wv, _, semi, _, _ = bufs[b]
        pltpu.make_async_copy(src_hbm.at[pl.ds(0, 1)], srcv, semi).wait()
        pltpu.make_async_copy(dst_hbm.at[pl.ds(0, 1)], dstv, semi).wait()
        pltpu.make_async_copy(w_hbm.at[pl.ds(0, WROWS)], wv, semi).wait()

    def gather_issue(b):
        srcv, _, _, _, rows, _, semg, _ = bufs[b]
        pltpu.async_copy(x_hbm.at[srcv.at[0, pl.ds(0, 64)]],
                         rows.at[pl.ds(0, 64)], semg)
        pltpu.async_copy(x_hbm.at[srcv.at[0, pl.ds(64, 64)]],
                         rows.at[pl.ds(64, 64)], semg)

    def gather_wait(b):
        srcv, _, _, _, rows, _, semg, _ = bufs[b]
        pltpu.make_async_copy(x_hbm.at[srcv.at[0, pl.ds(0, 64)]],
                              rows.at[pl.ds(0, 64)], semg).wait()
        pltpu.make_async_copy(x_hbm.at[srcv.at[0, pl.ds(64, 64)]],
                              rows.at[pl.ds(64, 64)], semg).wait()

    def scatter_issue(b):
        _, _, dstx, _, rows, _, _, sems = bufs[b]
        pltpu.async_copy(rows, accum.at[dstx.at[0]], sems, add=True)

    def scatter_wait(b):
        _, _, dstx, _, rows, _, _, sems = bufs[b]
        pltpu.make_async_copy(rows, accum.at[dstx.at[0]], sems).wait()

    def _zero_row(i, c):
        for j in range(D // 16):
            rows0[i, pl.ds(j * 16, 16)] = jnp.zeros((16,), jnp.float32)
        return c
    lax.fori_loop(0, CHUNK, _zero_row, 0)
    base = sid * ROWS_PT
    for t in range(ROWS_PT // CHUNK):
        pltpu.sync_copy(rows0.at[pl.ds(0, CHUNK)],
                        accum.at[pl.ds(base + t * CHUNK, CHUNK)])
    rem = ROWS_PT % CHUNK
    pltpu.sync_copy(rows0.at[pl.ds(0, rem)],
                    accum.at[pl.ds(base + ROWS_PT - rem, rem)])
    plsc.subcore_barrier()

    idx_issue(0, 0)
    idx_wait(0)
    gather_issue(0)
    idx_issue(1, 1)

    def _super(k2, c):
        for j in range(2):
            k = k2 * 2 + j
            srcv, dstv, dstx, wv, rows, semi, semg, sems = bufs[j]
            b1 = 1 - j
            gather_wait(j)
            idx_wait(b1)

            @pl.when(k >= 1)
            def _():
                scatter_wait(b1)
            gather_issue(b1)

            def _grp(g, cc):
                w16 = wv[g]
                for r in range(16):
                    ws = w16[r]
                    row = g * 16 + r
                    for jj in range(D // 16):
                        rows[row, pl.ds(jj * 16, 16)] = (
                            rows[row, pl.ds(jj * 16, 16)] * ws)
                return cc
            lax.fori_loop(0, WROWS, _grp, 0)

            for jj in range(128 // 16):
                dstx[0, pl.ds(jj * 16, 16)] = dstv[0, pl.ds(jj * 16, 16)]
            scatter_issue(j)
            idx_issue(k + 2, j)
        return c
    lax.fori_loop(0, N_CHUNKS // 2, _super, 0)

    gather_wait(0)
    idx_wait(1)
    scatter_wait(1)

    plsc.subcore_barrier()
    pltpu.sync_copy(accum.at[pl.ds(base, ROWS_PT)],
                    out_hbm.at[cid, pl.ds(base, ROWS_PT)])


@jax.jit
def _sc_spmm_call(src2d, dst2d, w16d, x):
    mesh = plsc.VectorSubcoreMesh(core_axis_name="c", subcore_axis_name="s",
                                  num_cores=NC, num_subcores=NS)
    buf_scratch = []
    for _ in range(2):
        buf_scratch += [
            pltpu.VMEM((1, 128), jnp.int32),
            pltpu.VMEM((1, 128), jnp.int32),
            pltpu.VMEM((1, 128), jnp.int32),
            pltpu.VMEM((WROWS, 16), jnp.float32),
            pltpu.VMEM((CHUNK, D), jnp.float32),
            pltpu.SemaphoreType.DMA,
            pltpu.SemaphoreType.DMA,
            pltpu.SemaphoreType.DMA,
        ]
    f = pl.kernel(
        _spmm_kernel,
        out_type=jax.ShapeDtypeStruct((NC, N_PAD, D), jnp.float32),
        mesh=mesh,
        scratch_types=[pltpu.VMEM_SHARED((N_PAD, D), jnp.float32)]
        + buf_scratch,
    )
    return f(src2d, dst2d, w16d, x)


def _bn1_body(p_ref, g_ref, b_ref, o_ref):
    x = p_ref[0, :N, :] + p_ref[1, :N, :]
    inv_n = jnp.float32(1.0 / N)
    mean = jnp.sum(x, axis=0, keepdims=True) * inv_n
    msq = jnp.sum(x * x, axis=0, keepdims=True) * inv_n
    var = msq - mean * mean
    inv = lax.rsqrt(var + EPS) * g_ref[...]
    o_ref[...] = (x - mean) * inv + b_ref[...]


def _bn2_body(p_ref, x1_ref, g_ref, b_ref, o_ref):
    x = p_ref[0, :N, :] + p_ref[1, :N, :]
    inv_n = jnp.float32(1.0 / N)
    mean = jnp.sum(x, axis=0, keepdims=True) * inv_n
    msq = jnp.sum(x * x, axis=0, keepdims=True) * inv_n
    var = msq - mean * mean
    inv = lax.rsqrt(var + EPS) * g_ref[...]
    y = (x - mean) * inv + b_ref[...]
    o_ref[...] = (x1_ref[...] + y) * jnp.float32(0.5)


@jax.jit
def _bn1(partials, gamma, beta):
    return pl.pallas_call(
        _bn1_body,
        out_shape=jax.ShapeDtypeStruct((N, D), jnp.float32),
    )(partials, gamma.reshape(1, D), beta.reshape(1, D))


@jax.jit
def _bn2(partials, x1, gamma, beta):
    return pl.pallas_call(
        _bn2_body,
        out_shape=jax.ShapeDtypeStruct((N, D), jnp.float32),
    )(partials, x1, gamma.reshape(1, D), beta.reshape(1, D))


def kernel(edge_index, edge_weight, W, gamma1, beta1, gamma2, beta2):
    src = edge_index[0].astype(jnp.int32)
    dst = edge_index[1].astype(jnp.int32)
    w = edge_weight.astype(jnp.float32)
    pad = E_PAD - E
    src = jnp.concatenate([src, jnp.zeros((pad,), jnp.int32)])
    dst = jnp.concatenate([dst, jnp.zeros((pad,), jnp.int32)])
    w = jnp.concatenate([w, jnp.zeros((pad,), jnp.float32)])
    src2d = src.reshape(E_PAD // 128, 128)
    dst2d = dst.reshape(E_PAD // 128, 128)
    w16d = w.reshape(E_PAD // 16, 16)

    p1 = _sc_spmm_call(src2d, dst2d, w16d, W)
    x1 = _bn1(p1, gamma1, beta1)
    p2 = _sc_spmm_call(src2d, dst2d, w16d, x1)
    return _bn2(p2, x1, gamma2, beta2)
